# baseline (speedup 1.0000x reference)
import functools

import jax
import jax.numpy as jnp
from jax.experimental import pallas as pl
from jax.experimental.pallas import tpu as pltpu


def _se_body(x_ref, p_ref, o_ref, *, inv_hw, C):
    x = x_ref[...]
    s = jnp.sum(x, axis=2, keepdims=True)
    m = jnp.max(x, axis=2, keepdims=True)
    avg = s * inv_hw

    w1a = p_ref[0:C, :]
    w1m = p_ref[C:2 * C, :]
    b1c = p_ref[2 * C:3 * C, :]
    w2g = p_ref[3 * C:4 * C, :]
    w2b = p_ref[4 * C:5 * C, :]
    b2gc = p_ref[5 * C:6 * C, :]
    b2bc = p_ref[6 * C:7 * C, :]

    t = avg * w1a + m * w1m + b1c
    h = jnp.sum(t, axis=1, keepdims=True)
    h = h * jnp.tanh(jax.nn.softplus(h))

    gam = jnp.sum(w2g * h + b2gc, axis=2, keepdims=True)
    bet = jnp.sum(w2b * h + b2bc, axis=2, keepdims=True)
    scale = jax.nn.sigmoid(gam)

    o_ref[...] = (scale * x + bet).astype(o_ref.dtype)


def kernel(x_nchw, w1, b1, w2, b2):
    B, C, H, W = x_nchw.shape
    HW = H * W
    hidden = w1.shape[0]
    x = x_nchw.reshape(B, C, HW)
    f32 = jnp.float32

    w1a = w1[:, :C].T.astype(f32)
    w1m = w1[:, C:].T.astype(f32)
    b1c = jnp.broadcast_to(b1.astype(f32)[None, :] / C, (C, hidden))
    w2g = w2[:C, :].astype(f32)
    w2b = w2[C:, :].astype(f32)
    b2gc = jnp.broadcast_to(b2[:C, None].astype(f32) / hidden, (C, hidden))
    b2bc = jnp.broadcast_to(b2[C:, None].astype(f32) / hidden, (C, hidden))
    pack = jnp.concatenate([w1a, w1m, b1c, w2g, w2b, b2gc, b2bc], axis=0)

    per_image = C * HW * x.dtype.itemsize
    bt = 1
    for d in range(1, B + 1):
        if B % d == 0 and 4 * d * per_image <= 48 * 2**20 and B // d >= 2:
            bt = d

    body = functools.partial(_se_body, inv_hw=1.0 / HW, C=C)
    out = pl.pallas_call(
        body,
        out_shape=jax.ShapeDtypeStruct((B, C, HW), x.dtype),
        grid=(B // bt,),
        in_specs=[
            pl.BlockSpec((bt, C, HW), lambda i: (i, 0, 0)),
            pl.BlockSpec((7 * C, hidden), lambda i: (0, 0)),
        ],
        out_specs=pl.BlockSpec((bt, C, HW), lambda i: (i, 0, 0)),
        compiler_params=pltpu.CompilerParams(
            dimension_semantics=("arbitrary",),
            vmem_limit_bytes=64 * 2**20,
        ),
    )(x, pack)

    return out.reshape(B, C, H, W)
